# Initial kernel scaffold; baseline (speedup 1.0000x reference)
import jax
import jax.numpy as jnp
from jax.experimental import pallas as pl


def kernel(edge_index, batch, sphex, W_conv, b_conv, W_lin, b_lin):
    raise NotImplementedError("write your pallas kernel here")


# speedup vs baseline: 5.2936x; 5.2936x over previous
import functools

import jax
import jax.numpy as jnp
from jax import lax
from jax.experimental import pallas as pl
from jax.experimental.pallas import tpu as pltpu
from jax.experimental.pallas import tpu_sc as plsc

N = 10000
E = 160000
D = 256
NNB = 16

NC = 2
NS = 16
NW = NC * NS

NPAD = 10240
NODES_PER_W = NPAD // NW

DEG_CHUNK = 2000

@functools.cache
def _sc_mesh():
    return plsc.VectorSubcoreMesh(
        core_axis_name="c", subcore_axis_name="s",
        num_cores=NC, num_subcores=NS)


@functools.cache
def _build_deg_kernel():
    return pl.kernel(
        _deg_body,
        out_type=jax.ShapeDtypeStruct((NPAD,), jnp.float32),
        mesh=_sc_mesh(),
        scratch_types=[
            pltpu.VMEM((DEG_CHUNK,), jnp.int32),
            pltpu.VMEM((NODES_PER_W,), jnp.float32),
        ],
        compiler_params=pltpu.CompilerParams(needs_layout_passes=False),
    )


def _deg_body(col_hbm, deg_hbm, colbuf, hist):
    c = lax.axis_index("c")
    s = lax.axis_index("s")
    wid = c * NS + s
    lo = wid * NODES_PER_W

    zero16 = jnp.zeros((16,), jnp.float32)
    for i in range(NODES_PER_W // 16):
        hist[pl.ds(i * 16, 16)] = zero16

    one16 = jnp.ones((16,), jnp.float32)

    def chunk_body(k, carry):
        pltpu.sync_copy(col_hbm.at[pl.ds(k * DEG_CHUNK, DEG_CHUNK)], colbuf)

        def vec_body(v, carry2):
            cv = colbuf[pl.ds(v * 16, 16)]
            loc = cv - lo
            m = (loc >= 0) & (loc < NODES_PER_W)
            locc = jnp.clip(loc, 0, NODES_PER_W - 1)
            plsc.addupdate_scatter(hist, [locc], one16, mask=m)
            return carry2

        return lax.fori_loop(0, DEG_CHUNK // 16, vec_body, carry)

    lax.fori_loop(0, E // DEG_CHUNK, chunk_body, 0)
    pltpu.sync_copy(hist, deg_hbm.at[pl.ds(lo, NODES_PER_W)])


BN = 320


def _dense_body(sph_ref, wc_ref, wl_ref, bl_ref, deg_ref, xs_ref, mi_ref,
                sum_ref):
    b = pl.program_id(0)
    sph = sph_ref[...]
    sin = jnp.sin(sph)
    cos = jnp.cos(sph)
    ones_col = jnp.ones((BN, 1), jnp.float32)
    p = jnp.concatenate([ones_col, sin], axis=1)
    k = 1
    while k < D:
        shifted = jnp.concatenate(
            [jnp.ones((BN, k), jnp.float32), p[:, : D - k]], axis=1)
        p = p * shifted
        k *= 2
    cosp = jnp.concatenate([cos, ones_col], axis=1)
    gex = p * cosp
    gex = jnp.where(gex != gex, 0.0, gex)

    wc = wc_ref[...]
    wl = wl_ref[...]
    dn = (((1,), (1,)), ((), ()))
    x = lax.dot_general(gex, wc, dn, preferred_element_type=jnp.float32)
    mi = lax.dot_general(gex, wl, dn, preferred_element_type=jnp.float32)
    mi_ref[...] = mi + bl_ref[...]

    deg = deg_ref[...]
    dis = jnp.where(deg > 0, lax.rsqrt(jnp.maximum(deg, 1e-12)), 0.0)
    xs_ref[...] = x * dis

    rowid = b * BN + lax.broadcasted_iota(jnp.int32, (BN, 1), 0)
    gm = jnp.where(rowid < N, gex, 0.0)

    @pl.when(b == 0)
    def _():
        sum_ref[...] = jnp.zeros_like(sum_ref)

    sum_ref[...] += jnp.sum(gm, axis=0, keepdims=True)


_dense = pl.pallas_call(
    _dense_body,
    grid=(NPAD // BN,),
    in_specs=[
        pl.BlockSpec((BN, D - 1), lambda b: (b, 0)),
        pl.BlockSpec((D, D), lambda b: (0, 0)),
        pl.BlockSpec((D, D), lambda b: (0, 0)),
        pl.BlockSpec((1, D), lambda b: (0, 0)),
        pl.BlockSpec((BN, 1), lambda b: (b, 0)),
    ],
    out_specs=[
        pl.BlockSpec((BN, D), lambda b: (b, 0)),
        pl.BlockSpec((BN, D), lambda b: (b, 0)),
        pl.BlockSpec((1, D), lambda b: (0, 0)),
    ],
    out_shape=[
        jax.ShapeDtypeStruct((NPAD, D), jnp.float32),
        jax.ShapeDtypeStruct((NPAD, D), jnp.float32),
        jax.ShapeDtypeStruct((1, D), jnp.float32),
    ],
)


EPW = E // NS
HALF = N // NC
ACC_ROWS = 5120
SCAN = 2000
RB = 64
FLAT = EPW + 512
DUMMY_ROW = N


@functools.cache
def _build_scatter_kernel():
    return pl.kernel(
        _scatter_body,
        out_type=jax.ShapeDtypeStruct((N, 16, 16), jnp.float32),
        mesh=_sc_mesh(),
        scratch_types=[
            pltpu.VMEM((SCAN,), jnp.int32),
            pltpu.VMEM((SCAN,), jnp.int32),
            pltpu.VMEM((FLAT,), jnp.int32),
            pltpu.VMEM((FLAT,), jnp.int32),
            pltpu.VMEM((RB,), jnp.int32),
            pltpu.VMEM((RB,), jnp.int32),
            pltpu.VMEM((RB, 16, 16), jnp.float32),
            pltpu.VMEM_SHARED((ACC_ROWS, 16, 16), jnp.float32),
            pltpu.SemaphoreType.DMA,
        ],
        compiler_params=pltpu.CompilerParams(
            needs_layout_passes=False, use_tc_tiling_on_sc=False),
    )


def _scatter_body(row_hbm, col_hbm, xs_hbm, out_hbm, rowbuf, colbuf,
                  flat_r, flat_c, ridx, cidx, rows_buf, acc, sem):
    c = lax.axis_index("c")
    s = lax.axis_index("s")
    lo = c * HALF
    base_e = s * EPW

    zero16 = jnp.zeros((16,), jnp.float32)

    def zrow(i, carry):
        def zv(j, carry2):
            rows_buf[i, j] = zero16
            return carry2
        return lax.fori_loop(0, D // 16, zv, carry)

    lax.fori_loop(0, RB, zrow, 0)
    abase = s * (ACC_ROWS // NS)
    for t in range((ACC_ROWS // NS) // RB):
        pltpu.sync_copy(rows_buf, acc.at[pl.ds(abase + t * RB, RB)])
    plsc.subcore_barrier()

    def scan_chunk(k, cnt):
        eb = base_e + k * SCAN
        pltpu.sync_copy(row_hbm.at[pl.ds(eb, SCAN)], rowbuf)
        pltpu.sync_copy(col_hbm.at[pl.ds(eb, SCAN)], colbuf)

        def vec(v, cnt2):
            cv = colbuf[pl.ds(v * 16, 16)]
            rv = rowbuf[pl.ds(v * 16, 16)]
            loc = cv - lo
            m = (loc >= 0) & (loc < HALF)
            plsc.store_compressed(flat_r.at[pl.ds(cnt2, 16)], rv, mask=m)
            plsc.store_compressed(flat_c.at[pl.ds(cnt2, 16)], loc, mask=m)
            return cnt2 + jnp.sum(m.astype(jnp.int32))

        return lax.fori_loop(0, SCAN // 16, vec, cnt)

    cnt = lax.fori_loop(0, EPW // SCAN, scan_chunk, jnp.int32(0))

    dummy_r = jnp.full((16,), DUMMY_ROW, jnp.int32)
    dummy_c = jnp.zeros((16,), jnp.int32)
    for i in range(RB // 16):
        flat_r[pl.ds(cnt + i * 16, 16)] = dummy_r
        flat_c[pl.ds(cnt + i * 16, 16)] = dummy_c

    rounds = (cnt + RB - 1) // RB

    def round_body(r, carry):
        off = r * RB
        for j in range(RB // 16):
            ridx[pl.ds(j * 16, 16)] = flat_r[pl.ds(off + j * 16, 16)]
            cidx[pl.ds(j * 16, 16)] = flat_c[pl.ds(off + j * 16, 16)]
        pltpu.async_copy(xs_hbm.at[ridx], rows_buf, sem).wait()
        pltpu.sync_copy(rows_buf, acc.at[cidx], add=True)
        return carry

    lax.fori_loop(0, rounds, round_body, 0)
    plsc.subcore_barrier()

    out_base = c * HALF
    per = 312
    pltpu.sync_copy(acc.at[pl.ds(s * per, per)],
                    out_hbm.at[pl.ds(out_base + s * per, per)])

    @pl.when(s == NS - 1)
    def _():
        pltpu.sync_copy(acc.at[pl.ds(NS * per, 8)],
                        out_hbm.at[pl.ds(out_base + NS * per, 8)])


def _logz_body(sum_ref, wc_ref, wl_ref, out_ref):
    mean = sum_ref[...] / float(N)
    wc = wc_ref[...]
    wl = wl_ref[...]
    dn = (((1,), (1,)), ((), ()))
    u1 = lax.dot_general(mean, NNB * wc + 2.0 * wl, dn,
                         preferred_element_type=jnp.float32)
    g = jnp.sqrt(jnp.sum(u1 * u1, axis=1, keepdims=True))
    u2 = lax.dot_general(mean, wl + 0.5 * NNB * wc, dn,
                         preferred_element_type=jnp.float32)
    z_mean = -float(N) * jnp.sum(mean * u2, axis=1, keepdims=True)
    g_hi = jnp.maximum(g, 20.0)
    g_lo = jnp.minimum(g, 20.0)
    z_hi = float(N) * (g_hi - jnp.log(g_hi))
    z_lo = float(N) * jnp.log(
        (jnp.exp(g_lo) - jnp.exp(-g_lo)) / jnp.maximum(g_lo, 1e-30))
    out_ref[...] = z_mean + jnp.where(g > 20.0, z_hi, z_lo)


_logz = pl.pallas_call(
    _logz_body,
    out_shape=jax.ShapeDtypeStruct((1, 1), jnp.float32),
)


BN2 = 400


def _finish_body(raw_ref, deg_ref, bc_ref, out_ref):
    deg = deg_ref[...]
    dis = jnp.where(deg > 0, lax.rsqrt(jnp.maximum(deg, 1e-12)), 0.0)
    out_ref[...] = raw_ref[...] * dis + bc_ref[...]


_finish = pl.pallas_call(
    _finish_body,
    grid=(N // BN2,),
    in_specs=[
        pl.BlockSpec((BN2, D), lambda b: (b, 0)),
        pl.BlockSpec((BN2, 1), lambda b: (b, 0)),
        pl.BlockSpec((1, D), lambda b: (0, 0)),
    ],
    out_specs=pl.BlockSpec((BN2, D), lambda b: (b, 0)),
    out_shape=jax.ShapeDtypeStruct((N, D), jnp.float32),
)


def kernel(edge_index, batch, sphex, W_conv, b_conv, W_lin, b_lin):
    row = edge_index[0]
    col = edge_index[1]
    sph_pad = jnp.pad(sphex, ((0, NPAD - N), (0, 0)))

    deg = _build_deg_kernel()(col)
    deg2 = deg.reshape(NPAD, 1)
    xs, mi, sums = _dense(sph_pad, W_conv, W_lin, b_lin.reshape(1, D), deg2)
    msg_raw = _build_scatter_kernel()(
        row, col, xs.reshape(NPAD, 16, 16)).reshape(N, D)
    logz = _logz(sums, W_conv, W_lin)
    msg = _finish(msg_raw, deg2[:N], b_conv.reshape(1, D))
    return (msg, mi[:N], logz)


# speedup vs baseline: 7.4275x; 1.4031x over previous
import functools

import jax
import jax.numpy as jnp
from jax import lax
from jax.experimental import pallas as pl
from jax.experimental.pallas import tpu as pltpu
from jax.experimental.pallas import tpu_sc as plsc

N = 10000
E = 160000
D = 256
NNB = 16

NC = 2
NS = 16
NW = NC * NS

NPAD = 10240
NODES_PER_W = NPAD // NW

EDGES_PER_W = E // NW
DEG_STAGE = EDGES_PER_W + 8
NR16 = NPAD // 16

@functools.cache
def _sc_mesh():
    return plsc.VectorSubcoreMesh(
        core_axis_name="c", subcore_axis_name="s",
        num_cores=NC, num_subcores=NS)


@functools.cache
def _build_deg_kernel():
    return pl.kernel(
        _deg_body,
        out_type=jax.ShapeDtypeStruct((NC, NR16, 16), jnp.float32),
        mesh=_sc_mesh(),
        scratch_types=[
            pltpu.VMEM((DEG_STAGE,), jnp.int32),
            pltpu.VMEM((NR16, 16), jnp.float32),
            pltpu.VMEM((NR16 // NS, 16), jnp.float32),
            pltpu.VMEM((128,), jnp.int32),
            pltpu.VMEM_SHARED((NR16, 16), jnp.float32),
        ],
        compiler_params=pltpu.CompilerParams(
            needs_layout_passes=False, use_tc_tiling_on_sc=False),
    )


def _deg_body(col_hbm, deg_hbm, colbuf, hist, zbuf, rdi, deg_sp):
    c = lax.axis_index("c")
    s = lax.axis_index("s")
    wid = c * NS + s

    zero16 = jnp.zeros((16,), jnp.float32)
    for i in range(NR16 // NS):
        zbuf[i] = zero16
    pltpu.sync_copy(zbuf, deg_sp.at[pl.ds(s * (NR16 // NS), NR16 // NS)])

    def zh(i, carry):
        hist[i] = zero16
        return carry

    lax.fori_loop(0, NR16, zh, 0)

    pltpu.sync_copy(col_hbm.at[pl.ds(wid * EDGES_PER_W, DEG_STAGE)], colbuf)

    one16 = jnp.ones((16,), jnp.float32)
    lane = lax.iota(jnp.int32, 16)

    def vec_body(v, carry):
        cv = colbuf[pl.ds(v * 16, 16)]
        m = (v * 16 + lane) < EDGES_PER_W
        plsc.addupdate_scatter(hist, [cv >> 4, cv & 15], one16, mask=m)
        return carry

    lax.fori_loop(0, DEG_STAGE // 16, vec_body, 0)

    plsc.subcore_barrier()
    for r in range(NR16 // 128):
        for j in range(8):
            rdi[pl.ds(j * 16, 16)] = r * 128 + j * 16 + lane
        pltpu.sync_copy(hist.at[pl.ds(r * 128, 128)], deg_sp.at[rdi],
                        add=True)
    plsc.subcore_barrier()
    pltpu.sync_copy(deg_sp.at[pl.ds(s * (NR16 // NS), NR16 // NS)],
                    deg_hbm.at[c, pl.ds(s * (NR16 // NS), NR16 // NS)])


BN = 320


def _dense_body(sph_ref, wc_ref, wl_ref, bl_ref, dega_ref, degb_ref,
                xs_ref, mi_ref, sum_ref):
    b = pl.program_id(0)
    sph = sph_ref[...]
    sin = jnp.sin(sph)
    cos = jnp.cos(sph)
    ones_col = jnp.ones((BN, 1), jnp.float32)
    p = jnp.concatenate([ones_col, sin], axis=1)
    k = 1
    while k < D:
        shifted = jnp.concatenate(
            [jnp.ones((BN, k), jnp.float32), p[:, : D - k]], axis=1)
        p = p * shifted
        k *= 2
    cosp = jnp.concatenate([cos, ones_col], axis=1)
    gex = p * cosp
    gex = jnp.where(gex != gex, 0.0, gex)

    wc = wc_ref[...]
    wl = wl_ref[...]
    dn = (((1,), (1,)), ((), ()))
    x = lax.dot_general(gex, wc, dn, preferred_element_type=jnp.float32)
    mi = lax.dot_general(gex, wl, dn, preferred_element_type=jnp.float32)
    mi_ref[...] = mi + bl_ref[...]

    deg = dega_ref[...] + degb_ref[...]
    dis = jnp.where(deg > 0, lax.rsqrt(jnp.maximum(deg, 1e-12)), 0.0)
    xs_ref[...] = x * dis

    rowid = b * BN + lax.broadcasted_iota(jnp.int32, (BN, 1), 0)
    gm = jnp.where(rowid < N, gex, 0.0)

    @pl.when(b == 0)
    def _():
        sum_ref[...] = jnp.zeros_like(sum_ref)

    sum_ref[...] += jnp.sum(gm, axis=0, keepdims=True)


_dense = pl.pallas_call(
    _dense_body,
    grid=(NPAD // BN,),
    in_specs=[
        pl.BlockSpec((BN, D - 1), lambda b: (b, 0)),
        pl.BlockSpec((D, D), lambda b: (0, 0)),
        pl.BlockSpec((D, D), lambda b: (0, 0)),
        pl.BlockSpec((1, D), lambda b: (0, 0)),
        pl.BlockSpec((BN, 1), lambda b: (b, 0)),
        pl.BlockSpec((BN, 1), lambda b: (b, 0)),
    ],
    out_specs=[
        pl.BlockSpec((BN, D), lambda b: (b, 0)),
        pl.BlockSpec((BN, D), lambda b: (b, 0)),
        pl.BlockSpec((1, D), lambda b: (0, 0)),
    ],
    out_shape=[
        jax.ShapeDtypeStruct((NPAD, D), jnp.float32),
        jax.ShapeDtypeStruct((NPAD, D), jnp.float32),
        jax.ShapeDtypeStruct((1, D), jnp.float32),
    ],
)


EPW = E // NS
HALF = N // NC
ACC_ROWS = 5120
SCAN = 2000
RB = 64
FLAT = SCAN + 2 * RB
DUMMY_ROW = N


@functools.cache
def _build_scatter_kernel():
    return pl.kernel(
        _scatter_body,
        out_type=jax.ShapeDtypeStruct((N, 16, 16), jnp.float32),
        mesh=_sc_mesh(),
        scratch_types=[
            pltpu.VMEM((SCAN,), jnp.int32),
            pltpu.VMEM((SCAN,), jnp.int32),
            pltpu.VMEM((FLAT,), jnp.int32),
            pltpu.VMEM((FLAT,), jnp.int32),
            pltpu.VMEM((RB,), jnp.int32),
            pltpu.VMEM((RB,), jnp.int32),
            pltpu.VMEM((RB,), jnp.int32),
            pltpu.VMEM((RB,), jnp.int32),
            pltpu.VMEM((RB, 16, 16), jnp.float32),
            pltpu.VMEM((RB, 16, 16), jnp.float32),
            pltpu.VMEM_SHARED((ACC_ROWS, 16, 16), jnp.float32),
            pltpu.SemaphoreType.DMA,
            pltpu.SemaphoreType.DMA,
        ],
        compiler_params=pltpu.CompilerParams(
            needs_layout_passes=False, use_tc_tiling_on_sc=False),
    )


def _scatter_body(row_hbm, col_hbm, xs_hbm, out_hbm, rowbuf, colbuf,
                  flat_r, flat_c, ridx0, cidx0, ridx1, cidx1,
                  rows_buf0, rows_buf1, acc, sem0, sem1):
    c = lax.axis_index("c")
    s = lax.axis_index("s")
    lo = c * HALF
    base_e = s * EPW
    ridx = (ridx0, ridx1)
    cidx = (cidx0, cidx1)
    rows_buf = (rows_buf0, rows_buf1)
    sem = (sem0, sem1)

    zero16 = jnp.zeros((16,), jnp.float32)

    def zrow(i, carry):
        def zv(j, carry2):
            rows_buf0[i, j] = zero16
            return carry2
        return lax.fori_loop(0, D // 16, zv, carry)

    lax.fori_loop(0, RB, zrow, 0)
    abase = s * (ACC_ROWS // NS)
    for t in range((ACC_ROWS // NS) // RB):
        pltpu.sync_copy(rows_buf0, acc.at[pl.ds(abase + t * RB, RB)])
    plsc.subcore_barrier()

    def fill_issue(r, b):
        off = r * RB
        for j in range(RB // 16):
            ridx[b][pl.ds(j * 16, 16)] = flat_r[pl.ds(off + j * 16, 16)]
            cidx[b][pl.ds(j * 16, 16)] = flat_c[pl.ds(off + j * 16, 16)]
        pltpu.async_copy(xs_hbm.at[ridx[b]], rows_buf[b], sem[b])

    def wait_scatter(b):
        pltpu.make_async_copy(xs_hbm.at[ridx[b]], rows_buf[b], sem[b]).wait()
        pltpu.sync_copy(rows_buf[b], acc.at[cidx[b]], add=True)

    def flush(cnt, final):
        if final:
            dummy_r = jnp.full((16,), DUMMY_ROW, jnp.int32)
            dummy_c = jnp.zeros((16,), jnp.int32)
            for i in range(RB // 16):
                flat_r[pl.ds(cnt + i * 16, 16)] = dummy_r
                flat_c[pl.ds(cnt + i * 16, 16)] = dummy_c
            full = (cnt + RB - 1) // RB
        else:
            full = cnt // RB

        @pl.when(full >= 1)
        def _():
            fill_issue(0, 0)

        @pl.when(full >= 2)
        def _():
            fill_issue(1, 1)

        def body(i, carry):
            r0 = 2 * i

            @pl.when(r0 < full)
            def _():
                wait_scatter(0)

                @pl.when(r0 + 2 < full)
                def _():
                    fill_issue(r0 + 2, 0)

            r1 = 2 * i + 1

            @pl.when(r1 < full)
            def _():
                wait_scatter(1)

                @pl.when(r1 + 2 < full)
                def _():
                    fill_issue(r1 + 2, 1)

            return carry

        lax.fori_loop(0, (full + 1) // 2, body, 0)
        if final:
            return jnp.int32(0)
        rem = cnt - full * RB
        src = full * RB
        for j in range(RB // 16):
            tr = flat_r[pl.ds(src + j * 16, 16)]
            tc = flat_c[pl.ds(src + j * 16, 16)]
            flat_r[pl.ds(j * 16, 16)] = tr
            flat_c[pl.ds(j * 16, 16)] = tc
        return rem

    cnt = jnp.int32(0)
    for k in range(EPW // SCAN):
        eb = base_e + k * SCAN
        pltpu.sync_copy(row_hbm.at[pl.ds(eb, SCAN)], rowbuf)
        pltpu.sync_copy(col_hbm.at[pl.ds(eb, SCAN)], colbuf)

        def vec(v, cnt2):
            cv = colbuf[pl.ds(v * 16, 16)]
            rv = rowbuf[pl.ds(v * 16, 16)]
            loc = cv - lo
            m = (loc >= 0) & (loc < HALF)
            plsc.store_compressed(flat_r.at[pl.ds(cnt2, 16)], rv, mask=m)
            plsc.store_compressed(flat_c.at[pl.ds(cnt2, 16)], loc, mask=m)
            return cnt2 + jnp.sum(m.astype(jnp.int32))

        cnt = lax.fori_loop(0, SCAN // 16, vec, cnt)
        cnt = flush(cnt, final=False)
    flush(cnt, final=True)
    plsc.subcore_barrier()

    out_base = c * HALF
    per = 312
    pltpu.sync_copy(acc.at[pl.ds(s * per, per)],
                    out_hbm.at[pl.ds(out_base + s * per, per)])

    @pl.when(s == NS - 1)
    def _():
        pltpu.sync_copy(acc.at[pl.ds(NS * per, 8)],
                        out_hbm.at[pl.ds(out_base + NS * per, 8)])


def _logz_body(sum_ref, wc_ref, wl_ref, out_ref):
    mean = sum_ref[...] / float(N)
    wc = wc_ref[...]
    wl = wl_ref[...]
    dn = (((1,), (1,)), ((), ()))
    u1 = lax.dot_general(mean, NNB * wc + 2.0 * wl, dn,
                         preferred_element_type=jnp.float32)
    g = jnp.sqrt(jnp.sum(u1 * u1, axis=1, keepdims=True))
    u2 = lax.dot_general(mean, wl + 0.5 * NNB * wc, dn,
                         preferred_element_type=jnp.float32)
    z_mean = -float(N) * jnp.sum(mean * u2, axis=1, keepdims=True)
    g_hi = jnp.maximum(g, 20.0)
    g_lo = jnp.minimum(g, 20.0)
    z_hi = float(N) * (g_hi - jnp.log(g_hi))
    z_lo = float(N) * jnp.log(
        (jnp.exp(g_lo) - jnp.exp(-g_lo)) / jnp.maximum(g_lo, 1e-30))
    out_ref[...] = z_mean + jnp.where(g > 20.0, z_hi, z_lo)


_logz = pl.pallas_call(
    _logz_body,
    out_shape=jax.ShapeDtypeStruct((1, 1), jnp.float32),
)


BN2 = 400


def _finish_body(raw_ref, dega_ref, degb_ref, bc_ref, out_ref):
    deg = dega_ref[...] + degb_ref[...]
    dis = jnp.where(deg > 0, lax.rsqrt(jnp.maximum(deg, 1e-12)), 0.0)
    out_ref[...] = raw_ref[...] * dis + bc_ref[...]


_finish = pl.pallas_call(
    _finish_body,
    grid=(N // BN2,),
    in_specs=[
        pl.BlockSpec((BN2, D), lambda b: (b, 0)),
        pl.BlockSpec((BN2, 1), lambda b: (b, 0)),
        pl.BlockSpec((BN2, 1), lambda b: (b, 0)),
        pl.BlockSpec((1, D), lambda b: (0, 0)),
    ],
    out_specs=pl.BlockSpec((BN2, D), lambda b: (b, 0)),
    out_shape=jax.ShapeDtypeStruct((N, D), jnp.float32),
)


def kernel(edge_index, batch, sphex, W_conv, b_conv, W_lin, b_lin):
    row = edge_index[0]
    col = edge_index[1]
    col_pad = jnp.pad(col, (0, 64))
    sph_pad = jnp.pad(sphex, ((0, NPAD - N), (0, 0)))

    degp = _build_deg_kernel()(col_pad).reshape(NC, NPAD)
    dega = degp[0].reshape(NPAD, 1)
    degb = degp[1].reshape(NPAD, 1)
    xs, mi, sums = _dense(sph_pad, W_conv, W_lin, b_lin.reshape(1, D),
                          dega, degb)
    msg_raw = _build_scatter_kernel()(
        row, col, xs.reshape(NPAD, 16, 16)).reshape(N, D)
    logz = _logz(sums, W_conv, W_lin)
    msg = _finish(msg_raw, dega[:N], degb[:N], b_conv.reshape(1, D))
    return (msg, mi[:N], logz)
